# baseline (speedup 1.0000x reference)
import functools

import jax
import jax.numpy as jnp
from jax import lax
from jax.experimental import pallas as pl
from jax.experimental.pallas import tpu as pltpu
from jax.experimental.pallas import tpu_sc as plsc

NUM_CLASSES = 1000000
EMBED_DIM = 64
BATCH = 16384

NC = 2
NS = 16
NW = NC * NS
B_PER_W = BATCH // NW


def _extract_row(slice_ref, lane, rows_ref, row_off):
    lane_vec = jnp.full((16,), lane, dtype=jnp.int32)
    for j in range(EMBED_DIM // 16):
        c_vec = lax.iota(jnp.int32, 16) + 16 * j
        vals = plsc.load_gather(slice_ref, [c_vec, lane_vec])
        rows_ref[pl.ds(row_off + 16 * j, 16)] = vals


def _sc_gather(table_t, class_index):
    mesh = plsc.VectorSubcoreMesh(core_axis_name="c", subcore_axis_name="s")

    @functools.partial(
        pl.kernel,
        mesh=mesh,
        out_type=jax.ShapeDtypeStruct((BATCH * EMBED_DIM,), jnp.float32),
        scratch_types=[
            pltpu.VMEM((B_PER_W,), jnp.int32),
            pltpu.VMEM((EMBED_DIM, 16), jnp.float32),
            pltpu.VMEM((EMBED_DIM, 16), jnp.float32),
            pltpu.VMEM((B_PER_W * EMBED_DIM,), jnp.float32),
            pltpu.SemaphoreType.DMA,
            pltpu.SemaphoreType.DMA,
        ],
        compiler_params=pltpu.CompilerParams(
            use_tc_tiling_on_sc=False, needs_layout_passes=False
        ),
    )
    def gather(table_hbm, idx_hbm, out_hbm, idx_v, buf0, buf1, rows_v, sem0, sem1):
        wid = lax.axis_index("s") * NC + lax.axis_index("c")
        base = wid * B_PER_W
        pltpu.sync_copy(idx_hbm.at[pl.ds(base, B_PER_W)], idx_v)

        def read_idx(j):
            grp = pl.multiple_of(
                lax.shift_left(lax.shift_right_logical(j, 4), 4), 16
            )
            vec = idx_v[pl.ds(grp, 16)]
            lane = lax.iota(jnp.int32, 16)
            sel = jnp.where(lane == (j & 15), vec, 0)
            return jnp.sum(sel)

        def fire(j, buf, sem):
            r = read_idx(j)
            q = pl.multiple_of(lax.shift_left(lax.shift_right_logical(r, 4), 4), 16)
            pltpu.async_copy(table_hbm.at[:, pl.ds(q, 16)], buf, sem)

        def drain(buf, sem):
            pltpu.make_async_copy(table_hbm.at[:, pl.ds(0, 16)], buf, sem).wait()

        fire(0, buf0, sem0)
        fire(1, buf1, sem1)

        def body(k, carry):
            r0 = 2 * k
            drain(buf0, sem0)
            _extract_row(buf0, read_idx(r0) & 15, rows_v, r0 * EMBED_DIM)
            fire(jnp.minimum(r0 + 2, B_PER_W - 1), buf0, sem0)
            r1 = 2 * k + 1
            drain(buf1, sem1)
            _extract_row(buf1, read_idx(r1) & 15, rows_v, r1 * EMBED_DIM)
            fire(jnp.minimum(r1 + 2, B_PER_W - 1), buf1, sem1)
            return carry

        lax.fori_loop(0, B_PER_W // 2, body, 0)
        drain(buf0, sem0)
        drain(buf1, sem1)
        pltpu.sync_copy(rows_v, out_hbm.at[pl.ds(base * EMBED_DIM, B_PER_W * EMBED_DIM)])

    return gather(table_t, class_index)


def _mlp_body(emb_ref, w1_ref, b1_ref, w2_ref, b2_ref, out_ref):
    x = emb_ref[...]
    h = jnp.dot(x, w1_ref[...], preferred_element_type=jnp.float32) + b1_ref[...]
    h = h * jax.nn.sigmoid(h)
    out_ref[...] = (
        jnp.dot(h, w2_ref[...], preferred_element_type=jnp.float32) + b2_ref[...]
    )


def _tc_mlp(emb, W1, b1, W2, b2):
    blk = 2048
    grid = (BATCH // blk,)
    return pl.pallas_call(
        _mlp_body,
        grid=grid,
        in_specs=[
            pl.BlockSpec((blk, EMBED_DIM), lambda i: (i, 0)),
            pl.BlockSpec((EMBED_DIM, EMBED_DIM), lambda i: (0, 0)),
            pl.BlockSpec((1, EMBED_DIM), lambda i: (0, 0)),
            pl.BlockSpec((EMBED_DIM, EMBED_DIM), lambda i: (0, 0)),
            pl.BlockSpec((1, EMBED_DIM), lambda i: (0, 0)),
        ],
        out_specs=pl.BlockSpec((blk, EMBED_DIM), lambda i: (i, 0)),
        out_shape=jax.ShapeDtypeStruct((BATCH, EMBED_DIM), jnp.float32),
        compiler_params=pltpu.CompilerParams(
            dimension_semantics=("parallel",),
        ),
    )(emb, W1, b1.reshape(1, EMBED_DIM), W2, b2.reshape(1, EMBED_DIM))


def kernel(class_index, table, W1, b1, W2, b2):
    emb_flat = _sc_gather(table.T, class_index.astype(jnp.int32))
    emb = emb_flat.reshape(BATCH, EMBED_DIM)
    return _tc_mlp(emb, W1, b1, W2, b2)


# speedup vs baseline: 19.4718x; 19.4718x over previous
import functools

import jax
import jax.numpy as jnp
from jax import lax
from jax.experimental import pallas as pl
from jax.experimental.pallas import tpu as pltpu
from jax.experimental.pallas import tpu_sc as plsc

NUM_CLASSES = 1000000
EMBED_DIM = 64
BATCH = 16384

NC = 2
NS = 16
NW = NC * NS
B_PER_W = BATCH // NW
NBUF = 4


def _extract_row(slice_ref, lane, rows_ref, row_off):
    lane_vec = jnp.full((16,), lane, dtype=jnp.int32)
    for j in range(EMBED_DIM // 16):
        c_vec = lax.iota(jnp.int32, 16) + 16 * j
        vals = plsc.load_gather(slice_ref, [c_vec, lane_vec])
        rows_ref[pl.ds(row_off + 16 * j, 16)] = vals


def _sc_gather(table_t, class_index):
    mesh = plsc.VectorSubcoreMesh(core_axis_name="c", subcore_axis_name="s")

    @functools.partial(
        pl.kernel,
        mesh=mesh,
        out_type=jax.ShapeDtypeStruct((BATCH * EMBED_DIM,), jnp.float32),
        scratch_types=[
            pltpu.VMEM((B_PER_W,), jnp.int32),
            [pltpu.VMEM((EMBED_DIM, 128), jnp.float32) for _ in range(NBUF)],
            pltpu.VMEM((B_PER_W * EMBED_DIM,), jnp.float32),
            [pltpu.SemaphoreType.DMA for _ in range(NBUF)],
        ],
        compiler_params=pltpu.CompilerParams(needs_layout_passes=False),
    )
    def gather(table_hbm, idx_hbm, out_hbm, idx_v, bufs, rows_v, sems):
        wid = lax.axis_index("s") * NC + lax.axis_index("c")
        base = wid * B_PER_W
        pltpu.sync_copy(idx_hbm.at[pl.ds(base, B_PER_W)], idx_v)

        def read_idx(j):
            grp = pl.multiple_of(
                lax.shift_left(lax.shift_right_logical(j, 4), 4), 16
            )
            vec = idx_v[pl.ds(grp, 16)]
            lane = lax.iota(jnp.int32, 16)
            sel = jnp.where(lane == (j & 15), vec, 0)
            return jnp.sum(sel)

        def fire(j, b):
            r = read_idx(j)
            q = pl.multiple_of(lax.shift_left(lax.shift_right_logical(r, 7), 7), 128)
            pltpu.async_copy(table_hbm.at[:, pl.ds(q, 128)], bufs[b], sems[b])

        def drain(b):
            pltpu.make_async_copy(
                table_hbm.at[:, pl.ds(0, 128)], bufs[b], sems[b]
            ).wait()

        for b in range(NBUF):
            fire(b, b)

        def body(k, carry):
            for b in range(NBUF):
                j = NBUF * k + b
                drain(b)
                _extract_row(bufs[b], read_idx(j) & 127, rows_v, j * EMBED_DIM)
                fire(jnp.minimum(j + NBUF, B_PER_W - 1), b)
            return carry

        lax.fori_loop(0, B_PER_W // NBUF, body, 0)
        for b in range(NBUF):
            drain(b)
        pltpu.sync_copy(rows_v, out_hbm.at[pl.ds(base * EMBED_DIM, B_PER_W * EMBED_DIM)])

    return gather(table_t, class_index)


def _mlp_body(emb_ref, w1_ref, b1_ref, w2_ref, b2_ref, out_ref):
    x = emb_ref[...]
    h = jnp.dot(x, w1_ref[...], preferred_element_type=jnp.float32) + b1_ref[...]
    h = h * jax.nn.sigmoid(h)
    out_ref[...] = (
        jnp.dot(h, w2_ref[...], preferred_element_type=jnp.float32) + b2_ref[...]
    )


def _tc_mlp(emb, W1, b1, W2, b2):
    blk = 2048
    grid = (BATCH // blk,)
    return pl.pallas_call(
        _mlp_body,
        grid=grid,
        in_specs=[
            pl.BlockSpec((blk, EMBED_DIM), lambda i: (i, 0)),
            pl.BlockSpec((EMBED_DIM, EMBED_DIM), lambda i: (0, 0)),
            pl.BlockSpec((1, EMBED_DIM), lambda i: (0, 0)),
            pl.BlockSpec((EMBED_DIM, EMBED_DIM), lambda i: (0, 0)),
            pl.BlockSpec((1, EMBED_DIM), lambda i: (0, 0)),
        ],
        out_specs=pl.BlockSpec((blk, EMBED_DIM), lambda i: (i, 0)),
        out_shape=jax.ShapeDtypeStruct((BATCH, EMBED_DIM), jnp.float32),
        compiler_params=pltpu.CompilerParams(
            dimension_semantics=("parallel",),
        ),
    )(emb, W1, b1.reshape(1, EMBED_DIM), W2, b2.reshape(1, EMBED_DIM))


def kernel(class_index, table, W1, b1, W2, b2):
    emb_flat = _sc_gather(table.T, class_index.astype(jnp.int32))
    emb = emb_flat.reshape(BATCH, EMBED_DIM)
    return _tc_mlp(emb, W1, b1, W2, b2)


# speedup vs baseline: 22.3319x; 1.1469x over previous
import functools

import jax
import jax.numpy as jnp
from jax import lax
from jax.experimental import pallas as pl
from jax.experimental.pallas import tpu as pltpu
from jax.experimental.pallas import tpu_sc as plsc

NUM_CLASSES = 1000000
EMBED_DIM = 64
BATCH = 16384

NC = 2
NS = 16
NW = NC * NS
B_PER_W = BATCH // NW
NBUF = 8


def _extract_row(slice_ref, lane, rows_ref, row_off):
    lane_vec = jnp.full((16,), lane, dtype=jnp.int32)
    for j in range(EMBED_DIM // 16):
        c_vec = lax.iota(jnp.int32, 16) + 16 * j
        vals = plsc.load_gather(slice_ref, [c_vec, lane_vec])
        rows_ref[pl.ds(row_off + 16 * j, 16)] = vals


def _sc_gather(table_t, class_index):
    mesh = plsc.VectorSubcoreMesh(core_axis_name="c", subcore_axis_name="s")

    @functools.partial(
        pl.kernel,
        mesh=mesh,
        out_type=jax.ShapeDtypeStruct((BATCH * EMBED_DIM,), jnp.float32),
        scratch_types=[
            pltpu.VMEM((B_PER_W,), jnp.int32),
            [pltpu.VMEM((EMBED_DIM, 128), jnp.float32) for _ in range(NBUF)],
            pltpu.VMEM((B_PER_W * EMBED_DIM,), jnp.float32),
            [pltpu.SemaphoreType.DMA for _ in range(NBUF)],
        ],
        compiler_params=pltpu.CompilerParams(needs_layout_passes=False),
    )
    def gather(table_hbm, idx_hbm, out_hbm, idx_v, bufs, rows_v, sems):
        wid = lax.axis_index("s") * NC + lax.axis_index("c")
        base = wid * B_PER_W
        pltpu.sync_copy(idx_hbm.at[pl.ds(base, B_PER_W)], idx_v)

        def read_idx(j):
            grp = pl.multiple_of(
                lax.shift_left(lax.shift_right_logical(j, 4), 4), 16
            )
            vec = idx_v[pl.ds(grp, 16)]
            lane = lax.iota(jnp.int32, 16)
            sel = jnp.where(lane == (j & 15), vec, 0)
            return jnp.sum(sel)

        def fire(j, b):
            r = read_idx(j)
            q = pl.multiple_of(lax.shift_left(lax.shift_right_logical(r, 7), 7), 128)
            pltpu.async_copy(table_hbm.at[:, pl.ds(q, 128)], bufs[b], sems[b])

        def drain(b):
            pltpu.make_async_copy(
                table_hbm.at[:, pl.ds(0, 128)], bufs[b], sems[b]
            ).wait()

        for b in range(NBUF):
            fire(b, b)

        def body(k, carry):
            for b in range(NBUF):
                j = NBUF * k + b
                drain(b)
                _extract_row(bufs[b], read_idx(j) & 127, rows_v, j * EMBED_DIM)
                fire(jnp.minimum(j + NBUF, B_PER_W - 1), b)
            return carry

        lax.fori_loop(0, B_PER_W // NBUF, body, 0)
        for b in range(NBUF):
            drain(b)
        pltpu.sync_copy(rows_v, out_hbm.at[pl.ds(base * EMBED_DIM, B_PER_W * EMBED_DIM)])

    return gather(table_t, class_index)


def _mlp_body(emb_ref, w1_ref, b1_ref, w2_ref, b2_ref, out_ref):
    x = emb_ref[...]
    h = jnp.dot(x, w1_ref[...], preferred_element_type=jnp.float32) + b1_ref[...]
    h = h * jax.nn.sigmoid(h)
    out_ref[...] = (
        jnp.dot(h, w2_ref[...], preferred_element_type=jnp.float32) + b2_ref[...]
    )


def _tc_mlp(emb, W1, b1, W2, b2):
    blk = 2048
    grid = (BATCH // blk,)
    return pl.pallas_call(
        _mlp_body,
        grid=grid,
        in_specs=[
            pl.BlockSpec((blk, EMBED_DIM), lambda i: (i, 0)),
            pl.BlockSpec((EMBED_DIM, EMBED_DIM), lambda i: (0, 0)),
            pl.BlockSpec((1, EMBED_DIM), lambda i: (0, 0)),
            pl.BlockSpec((EMBED_DIM, EMBED_DIM), lambda i: (0, 0)),
            pl.BlockSpec((1, EMBED_DIM), lambda i: (0, 0)),
        ],
        out_specs=pl.BlockSpec((blk, EMBED_DIM), lambda i: (i, 0)),
        out_shape=jax.ShapeDtypeStruct((BATCH, EMBED_DIM), jnp.float32),
        compiler_params=pltpu.CompilerParams(
            dimension_semantics=("parallel",),
        ),
    )(emb, W1, b1.reshape(1, EMBED_DIM), W2, b2.reshape(1, EMBED_DIM))


def kernel(class_index, table, W1, b1, W2, b2):
    emb_flat = _sc_gather(table.T, class_index.astype(jnp.int32))
    emb = emb_flat.reshape(BATCH, EMBED_DIM)
    return _tc_mlp(emb, W1, b1, W2, b2)


# speedup vs baseline: 22.6323x; 1.0134x over previous
import functools

import jax
import jax.numpy as jnp
from jax import lax
from jax.experimental import pallas as pl
from jax.experimental.pallas import tpu as pltpu
from jax.experimental.pallas import tpu_sc as plsc

NUM_CLASSES = 1000000
EMBED_DIM = 64
BATCH = 16384

NC = 2
NS = 16
NW = NC * NS
B_PER_W = BATCH // NW
NBUF = 10


def _extract_row(slice_ref, lane, rows_ref, row_off):
    lane_vec = jnp.full((16,), lane, dtype=jnp.int32)
    for j in range(EMBED_DIM // 16):
        c_vec = lax.iota(jnp.int32, 16) + 16 * j
        vals = plsc.load_gather(slice_ref, [c_vec, lane_vec])
        rows_ref[pl.ds(row_off + 16 * j, 16)] = vals


def _sc_gather(table_t, class_index):
    mesh = plsc.VectorSubcoreMesh(core_axis_name="c", subcore_axis_name="s")

    @functools.partial(
        pl.kernel,
        mesh=mesh,
        out_type=jax.ShapeDtypeStruct((BATCH * EMBED_DIM,), jnp.float32),
        scratch_types=[
            pltpu.VMEM((B_PER_W,), jnp.int32),
            [pltpu.VMEM((EMBED_DIM, 128), jnp.float32) for _ in range(NBUF)],
            pltpu.VMEM((B_PER_W * EMBED_DIM,), jnp.float32),
            [pltpu.SemaphoreType.DMA for _ in range(NBUF)],
        ],
        compiler_params=pltpu.CompilerParams(needs_layout_passes=False),
    )
    def gather(table_hbm, idx_hbm, out_hbm, idx_v, bufs, rows_v, sems):
        wid = lax.axis_index("s") * NC + lax.axis_index("c")
        base = wid * B_PER_W
        pltpu.sync_copy(idx_hbm.at[pl.ds(base, B_PER_W)], idx_v)

        def read_idx(j):
            grp = pl.multiple_of(
                lax.shift_left(lax.shift_right_logical(j, 4), 4), 16
            )
            vec = idx_v[pl.ds(grp, 16)]
            lane = lax.iota(jnp.int32, 16)
            sel = jnp.where(lane == (j & 15), vec, 0)
            return jnp.sum(sel)

        def fire(j, b):
            r = read_idx(j)
            q = pl.multiple_of(lax.shift_left(lax.shift_right_logical(r, 7), 7), 128)
            pltpu.async_copy(table_hbm.at[:, pl.ds(q, 128)], bufs[b], sems[b])

        def drain(b):
            pltpu.make_async_copy(
                table_hbm.at[:, pl.ds(0, 128)], bufs[b], sems[b]
            ).wait()

        for b in range(NBUF):
            fire(b, b)

        def body(k, carry):
            for b in range(NBUF):
                j = NBUF * k + b
                drain(b)
                _extract_row(bufs[b], read_idx(j) & 127, rows_v, j * EMBED_DIM)
                fire(jnp.minimum(j + NBUF, B_PER_W - 1), b)
            return carry

        lax.fori_loop(0, B_PER_W // NBUF, body, 0)
        for b in range(NBUF):
            drain(b)
        pltpu.sync_copy(rows_v, out_hbm.at[pl.ds(base * EMBED_DIM, B_PER_W * EMBED_DIM)])

    return gather(table_t, class_index)


def _mlp_body(emb_ref, w1_ref, b1_ref, w2_ref, b2_ref, out_ref):
    x = emb_ref[...]
    h = jnp.dot(x, w1_ref[...], preferred_element_type=jnp.float32) + b1_ref[...]
    h = h * jax.nn.sigmoid(h)
    out_ref[...] = (
        jnp.dot(h, w2_ref[...], preferred_element_type=jnp.float32) + b2_ref[...]
    )


def _tc_mlp(emb, W1, b1, W2, b2):
    blk = 2048
    grid = (BATCH // blk,)
    return pl.pallas_call(
        _mlp_body,
        grid=grid,
        in_specs=[
            pl.BlockSpec((blk, EMBED_DIM), lambda i: (i, 0)),
            pl.BlockSpec((EMBED_DIM, EMBED_DIM), lambda i: (0, 0)),
            pl.BlockSpec((1, EMBED_DIM), lambda i: (0, 0)),
            pl.BlockSpec((EMBED_DIM, EMBED_DIM), lambda i: (0, 0)),
            pl.BlockSpec((1, EMBED_DIM), lambda i: (0, 0)),
        ],
        out_specs=pl.BlockSpec((blk, EMBED_DIM), lambda i: (i, 0)),
        out_shape=jax.ShapeDtypeStruct((BATCH, EMBED_DIM), jnp.float32),
        compiler_params=pltpu.CompilerParams(
            dimension_semantics=("parallel",),
        ),
    )(emb, W1, b1.reshape(1, EMBED_DIM), W2, b2.reshape(1, EMBED_DIM))


def kernel(class_index, table, W1, b1, W2, b2):
    emb_flat = _sc_gather(table.T, class_index.astype(jnp.int32))
    emb = emb_flat.reshape(BATCH, EMBED_DIM)
    return _tc_mlp(emb, W1, b1, W2, b2)
